# Initial kernel scaffold; baseline (speedup 1.0000x reference)
import functools

import jax
import jax.numpy as jnp
from jax import lax
from jax.experimental import pallas as pl
from jax.experimental.pallas import tpu as pltpu
from jax.experimental.pallas import tpu_sc as plsc

BATCH = 16384
HIST = 50
DIM = 64
TOTAL = BATCH * HIST

_info = plsc.get_sparse_core_info()
NC = _info.num_cores
NS = _info.num_subcores
NW = NC * NS
ROWS_PER_W = TOTAL // NW

IDX_MINOR = 128
STREAMS_PER_CHUNK = 8
CHUNK = IDX_MINOR * STREAMS_PER_CHUNK
N_CHUNKS = ROWS_PER_W // CHUNK

_mesh = plsc.VectorSubcoreMesh(core_axis_name="c", subcore_axis_name="s")


@functools.partial(
    pl.kernel,
    out_type=jax.ShapeDtypeStruct((TOTAL, DIM), jnp.float32),
    mesh=_mesh,
    scratch_types=[
        pltpu.VMEM((STREAMS_PER_CHUNK, IDX_MINOR), jnp.int32),
        pltpu.VMEM((CHUNK, DIM), jnp.float32),
        pltpu.SemaphoreType.DMA,
    ],
)
def _gather_kernel(idx_hbm, table_hbm, out_hbm, idx_v, rows_v, sem):
    wid = lax.axis_index("s") * NC + lax.axis_index("c")
    row_base = wid * ROWS_PER_W
    idx_row_base = row_base // IDX_MINOR

    def chunk_body(g, carry):
        pltpu.sync_copy(
            idx_hbm.at[pl.ds(idx_row_base + g * STREAMS_PER_CHUNK,
                             STREAMS_PER_CHUNK)],
            idx_v,
        )
        descs = []
        for j in range(STREAMS_PER_CHUNK):
            descs.append(
                pltpu.async_copy(
                    table_hbm.at[idx_v.at[j]],
                    rows_v.at[pl.ds(j * IDX_MINOR, IDX_MINOR)],
                    sem,
                )
            )
        for d in descs:
            d.wait()
        pltpu.sync_copy(rows_v, out_hbm.at[pl.ds(row_base + g * CHUNK, CHUNK)])
        return carry

    lax.fori_loop(0, N_CHUNKS, chunk_body, 0)


def kernel(x, weight):
    idx = x.astype(jnp.int32).reshape(TOTAL // IDX_MINOR, IDX_MINOR)
    out = _gather_kernel(idx, weight)
    return out.reshape(BATCH, HIST, DIM)


# speedup vs baseline: 1.8553x; 1.8553x over previous
import functools

import jax
import jax.numpy as jnp
from jax import lax
from jax.experimental import pallas as pl
from jax.experimental.pallas import tpu as pltpu
from jax.experimental.pallas import tpu_sc as plsc

BATCH = 16384
HIST = 50
DIM = 64
TOTAL = BATCH * HIST

_info = plsc.get_sparse_core_info()
NC = _info.num_cores
NS = _info.num_subcores
NW = NC * NS
ROWS_PER_W = TOTAL // NW

IDX_MINOR = 128
STREAMS_PER_CHUNK = 8
CHUNK = IDX_MINOR * STREAMS_PER_CHUNK
N_CHUNKS = ROWS_PER_W // CHUNK

_mesh = plsc.VectorSubcoreMesh(core_axis_name="c", subcore_axis_name="s")


@functools.partial(
    pl.kernel,
    out_type=jax.ShapeDtypeStruct((TOTAL, DIM), jnp.float32),
    mesh=_mesh,
    compiler_params=pltpu.CompilerParams(use_tc_tiling_on_sc=False),
    scratch_types=[
        pltpu.VMEM((STREAMS_PER_CHUNK, IDX_MINOR), jnp.int32),
        pltpu.VMEM((CHUNK, DIM), jnp.float32),
        pltpu.SemaphoreType.DMA,
    ],
)
def _gather_kernel(idx_hbm, table_hbm, out_hbm, idx_v, rows_v, sem):
    wid = lax.axis_index("s") * NC + lax.axis_index("c")
    row_base = wid * ROWS_PER_W
    idx_row_base = row_base // IDX_MINOR

    def chunk_body(g, carry):
        idx_off = pl.multiple_of(
            idx_row_base + g * STREAMS_PER_CHUNK, STREAMS_PER_CHUNK)
        pltpu.sync_copy(
            idx_hbm.at[pl.ds(idx_off, STREAMS_PER_CHUNK)],
            idx_v,
        )
        descs = []
        for j in range(STREAMS_PER_CHUNK):
            descs.append(
                pltpu.async_copy(
                    table_hbm.at[idx_v.at[j]],
                    rows_v.at[pl.ds(j * IDX_MINOR, IDX_MINOR)],
                    sem,
                )
            )
        for d in descs:
            d.wait()
        pltpu.sync_copy(rows_v, out_hbm.at[pl.ds(row_base + g * CHUNK, CHUNK)])
        return carry

    lax.fori_loop(0, N_CHUNKS, chunk_body, 0)


def kernel(x, weight):
    idx = x.astype(jnp.int32).reshape(TOTAL // IDX_MINOR, IDX_MINOR)
    out = _gather_kernel(idx, weight)
    return out.reshape(BATCH, HIST, DIM)
